# Initial kernel scaffold; baseline (speedup 1.0000x reference)
import jax
import jax.numpy as jnp
from jax.experimental import pallas as pl


def kernel(inputs, token_table, position_table):
    raise NotImplementedError("write your pallas kernel here")


# speedup vs baseline: 3.4303x; 3.4303x over previous
import functools

import jax
import jax.numpy as jnp
from jax import lax
from jax.experimental import pallas as pl
from jax.experimental.pallas import tpu as pltpu
from jax.experimental.pallas import tpu_sc as plsc

_SEQ = 200
_D = 64
_LANES = 16
_SCALE = 8.0

_info = plsc.get_sparse_core_info()
_NC = _info.num_cores
_NS = _info.num_subcores
_NW = _NC * _NS


def _build_sc_gather(B: int, C: int):
    assert B % _NW == 0
    b_per_w = B // _NW
    assert b_per_w % C == 0 and C % _SEQ == 0
    n_chunks = b_per_w // C
    n_seq_per_chunk = C // _SEQ

    mesh = plsc.VectorSubcoreMesh(core_axis_name="c", subcore_axis_name="s")

    @functools.partial(
        pl.kernel,
        out_type=jax.ShapeDtypeStruct((B, _D), jnp.float32),
        mesh=mesh,
        scratch_types=[
            pltpu.VMEM((C,), jnp.int32),
            pltpu.VMEM((C, _D), jnp.float32),
            pltpu.VMEM((_SEQ, _D), jnp.float32),
            pltpu.SemaphoreType.DMA,
        ],
        compiler_params=pltpu.CompilerParams(use_tc_tiling_on_sc=False),
    )
    def sc_kernel(idx_hbm, ttab_hbm, ptab_hbm, out_hbm, idx_v, rows_v, pos_v, sem):
        wid = lax.axis_index("s") * _NC + lax.axis_index("c")
        base = wid * b_per_w
        pltpu.sync_copy(ptab_hbm, pos_v)

        def chunk_body(ci, carry):
            start = base + ci * C
            pltpu.sync_copy(idx_hbm.at[pl.ds(start, C)], idx_v)
            pltpu.async_copy(ttab_hbm.at[idx_v], rows_v, sem).wait()

            def row_body(r, c2):
                for q in range(n_seq_per_chunk):
                    rr = q * _SEQ + r
                    for j in range(_D // _LANES):
                        sl = pl.ds(j * _LANES, _LANES)
                        rows_v[rr, sl] = rows_v[rr, sl] * _SCALE + pos_v[r, sl]
                return c2

            lax.fori_loop(0, _SEQ, row_body, 0, unroll=False)
            pltpu.sync_copy(rows_v, out_hbm.at[pl.ds(start, C)])
            return carry

        lax.fori_loop(0, n_chunks, chunk_body, 0, unroll=False)

    return sc_kernel


@jax.jit
def kernel(inputs, token_table, position_table):
    batch, seq = inputs.shape
    d = token_table.shape[1]
    idx = inputs.reshape(-1).astype(jnp.int32)
    fn = _build_sc_gather(batch * seq, 400)
    out = fn(idx, token_table, position_table)
    return out.reshape(batch, seq, d)


# speedup vs baseline: 4.2041x; 1.2256x over previous
import functools

import jax
import jax.numpy as jnp
from jax import lax
from jax.experimental import pallas as pl
from jax.experimental.pallas import tpu as pltpu
from jax.experimental.pallas import tpu_sc as plsc

_SEQ = 200
_D = 64
_LANES = 16
_SCALE = 8.0

_info = plsc.get_sparse_core_info()
_NC = _info.num_cores
_NS = _info.num_subcores
_NW = _NC * _NS


def _build_sc_gather(B: int, C: int):
    assert B % _NW == 0
    b_per_w = B // _NW
    assert b_per_w % C == 0 and C % _SEQ == 0
    n_chunks = b_per_w // C
    assert n_chunks % 2 == 0 and n_chunks >= 4
    n_seq_per_chunk = C // _SEQ

    mesh = plsc.VectorSubcoreMesh(core_axis_name="c", subcore_axis_name="s")

    @functools.partial(
        pl.kernel,
        out_type=jax.ShapeDtypeStruct((B, _D), jnp.float32),
        mesh=mesh,
        scratch_types=[
            pltpu.VMEM((C,), jnp.int32),
            pltpu.VMEM((C,), jnp.int32),
            pltpu.VMEM((C, _D), jnp.float32),
            pltpu.VMEM((C, _D), jnp.float32),
            pltpu.VMEM((C, _D), jnp.float32),
            pltpu.VMEM((C, _D), jnp.float32),
            pltpu.VMEM((_SEQ, _D), jnp.float32),
            pltpu.SemaphoreType.DMA,
            pltpu.SemaphoreType.DMA,
            pltpu.SemaphoreType.DMA,
            pltpu.SemaphoreType.DMA,
            pltpu.SemaphoreType.DMA,
            pltpu.SemaphoreType.DMA,
        ],
        compiler_params=pltpu.CompilerParams(use_tc_tiling_on_sc=False),
    )
    def sc_kernel(idx_hbm, ttab_hbm, ptab_hbm, out_hbm,
                  idx0, idx1, rows0, rows1, stg0, stg1, pos_v,
                  sg0, sg1, so0, so1, si0, si1):
        idx_v = (idx0, idx1)
        rows_v = (rows0, rows1)
        stg_v = (stg0, stg1)
        sg = (sg0, sg1)
        so = (so0, so1)
        si = (si0, si1)

        wid = lax.axis_index("s") * _NC + lax.axis_index("c")
        base = wid * b_per_w
        pltpu.sync_copy(ptab_hbm, pos_v)

        def idx_copy(k, b):
            return pltpu.make_async_copy(
                idx_hbm.at[pl.ds(base + k * C, C)], idx_v[b], si[b])

        def gather(b):
            return pltpu.make_async_copy(ttab_hbm.at[idx_v[b]], rows_v[b], sg[b])

        def out_copy(k, b):
            return pltpu.make_async_copy(
                stg_v[b], out_hbm.at[pl.ds(base + k * C, C)], so[b])

        def compute(b):
            rows = rows_v[b]
            stg = stg_v[b]

            def row_body(r, c2):
                pvs = [pos_v[r, pl.ds(j * _LANES, _LANES)]
                       for j in range(_D // _LANES)]
                for q in range(n_seq_per_chunk):
                    rr = q * _SEQ + r
                    for j in range(_D // _LANES):
                        sl = pl.ds(j * _LANES, _LANES)
                        stg[rr, sl] = rows[rr, sl] * _SCALE + pvs[j]
                return c2

            lax.fori_loop(0, _SEQ, row_body, 0, unroll=False)

        pltpu.sync_copy(idx_hbm.at[pl.ds(base, C)], idx_v[0])
        gather(0).start()
        idx_copy(1, 1).start()

        def pair_body(ci, carry):
            k0 = ci * 2
            for b in (0, 1):
                k = k0 + b
                o = 1 - b
                gather(b).wait()
                @pl.when(k + 1 < n_chunks)
                def _():
                    idx_copy(k + 1, o).wait()
                    gather(o).start()
                @pl.when(k + 2 < n_chunks)
                def _():
                    idx_copy(k + 2, b).start()
                @pl.when(k >= 2)
                def _():
                    out_copy(k - 2, b).wait()
                compute(b)
                out_copy(k, b).start()
            return carry

        lax.fori_loop(0, n_chunks // 2, pair_body, 0, unroll=False)
        out_copy(n_chunks - 2, 0).wait()
        out_copy(n_chunks - 1, 1).wait()

    return sc_kernel


@jax.jit
def kernel(inputs, token_table, position_table):
    batch, seq = inputs.shape
    d = token_table.shape[1]
    idx = inputs.reshape(-1).astype(jnp.int32)
    fn = _build_sc_gather(batch * seq, 400)
    out = fn(idx, token_table, position_table)
    return out.reshape(batch, seq, d)


# speedup vs baseline: 4.2200x; 1.0038x over previous
import functools

import jax
import jax.numpy as jnp
from jax import lax
from jax.experimental import pallas as pl
from jax.experimental.pallas import tpu as pltpu
from jax.experimental.pallas import tpu_sc as plsc

_SEQ = 200
_D = 64
_LANES = 16
_SCALE = 8.0

_info = plsc.get_sparse_core_info()
_NC = _info.num_cores
_NS = _info.num_subcores
_NW = _NC * _NS


def _build_sc_gather(BATCH: int, C: int):
    B = BATCH * _SEQ
    assert B % _NW == 0
    b_per_w = B // _NW
    assert b_per_w % C == 0 and C % _SEQ == 0
    n_chunks = b_per_w // C
    assert n_chunks % 2 == 0 and n_chunks >= 4
    n_seq_per_chunk = C // _SEQ
    seq_per_w = b_per_w // _SEQ

    mesh = plsc.VectorSubcoreMesh(core_axis_name="c", subcore_axis_name="s")

    @functools.partial(
        pl.kernel,
        out_type=jax.ShapeDtypeStruct((BATCH, _SEQ, _D), jnp.float32),
        mesh=mesh,
        scratch_types=[
            pltpu.VMEM((C,), jnp.int32),
            pltpu.VMEM((C,), jnp.int32),
            pltpu.VMEM((C, _D), jnp.float32),
            pltpu.VMEM((C, _D), jnp.float32),
            pltpu.VMEM((n_seq_per_chunk, _SEQ, _D), jnp.float32),
            pltpu.VMEM((n_seq_per_chunk, _SEQ, _D), jnp.float32),
            pltpu.VMEM((_SEQ, _D), jnp.float32),
            pltpu.SemaphoreType.DMA,
            pltpu.SemaphoreType.DMA,
            pltpu.SemaphoreType.DMA,
            pltpu.SemaphoreType.DMA,
            pltpu.SemaphoreType.DMA,
            pltpu.SemaphoreType.DMA,
        ],
        compiler_params=pltpu.CompilerParams(use_tc_tiling_on_sc=False),
    )
    def sc_kernel(idx_hbm, ttab_hbm, ptab_hbm, out_hbm,
                  idx0, idx1, rows0, rows1, stg0, stg1, pos_v,
                  sg0, sg1, so0, so1, si0, si1):
        idx_v = (idx0, idx1)
        rows_v = (rows0, rows1)
        stg_v = (stg0, stg1)
        sg = (sg0, sg1)
        so = (so0, so1)
        si = (si0, si1)

        wid = lax.axis_index("s") * _NC + lax.axis_index("c")
        base = wid * b_per_w
        seq_base = wid * seq_per_w
        pltpu.sync_copy(ptab_hbm, pos_v)

        def idx_copy(k, b):
            return pltpu.make_async_copy(
                idx_hbm.at[pl.ds(base + k * C, C)], idx_v[b], si[b])

        def gather(b):
            return pltpu.make_async_copy(ttab_hbm.at[idx_v[b]], rows_v[b], sg[b])

        def out_copy(k, b):
            return pltpu.make_async_copy(
                stg_v[b],
                out_hbm.at[pl.ds(seq_base + k * n_seq_per_chunk, n_seq_per_chunk)],
                so[b])

        def compute(b):
            rows = rows_v[b]
            stg = stg_v[b]

            def row_body(r, c2):
                pvs = [pos_v[r, pl.ds(j * _LANES, _LANES)]
                       for j in range(_D // _LANES)]
                for q in range(n_seq_per_chunk):
                    rr = q * _SEQ + r
                    for j in range(_D // _LANES):
                        sl = pl.ds(j * _LANES, _LANES)
                        stg[q, r, sl] = rows[rr, sl] * _SCALE + pvs[j]
                return c2

            lax.fori_loop(0, _SEQ, row_body, 0, unroll=False)

        pltpu.sync_copy(idx_hbm.at[pl.ds(base, C)], idx_v[0])
        gather(0).start()
        idx_copy(1, 1).start()

        def pair_body(ci, carry):
            k0 = ci * 2
            for b in (0, 1):
                k = k0 + b
                o = 1 - b
                gather(b).wait()
                @pl.when(k + 1 < n_chunks)
                def _():
                    idx_copy(k + 1, o).wait()
                    gather(o).start()
                @pl.when(k + 2 < n_chunks)
                def _():
                    idx_copy(k + 2, b).start()
                @pl.when(k >= 2)
                def _():
                    out_copy(k - 2, b).wait()
                compute(b)
                out_copy(k, b).start()
            return carry

        lax.fori_loop(0, n_chunks // 2, pair_body, 0, unroll=False)
        out_copy(n_chunks - 2, 0).wait()
        out_copy(n_chunks - 1, 1).wait()

    return sc_kernel


@jax.jit
def kernel(inputs, token_table, position_table):
    batch, seq = inputs.shape
    idx = inputs.reshape(-1).astype(jnp.int32)
    fn = _build_sc_gather(batch, 400)
    return fn(idx, token_table, position_table)
